# baseline (speedup 1.0000x reference)
import functools

import jax
import jax.numpy as jnp
import numpy as np
from jax import lax
from jax.experimental import pallas as pl
from jax.experimental.pallas import tpu as pltpu
from jax.experimental.pallas import tpu_sc as plsc

_L = 512
_K = 32
_INT_MIN = np.int32(-(2**31))
_SC_CORES = 2
_SC_SUBCORES = 16
_CHUNKS = 1


def _topk_body(r, rowbase, k_ref, gm_ref, tw_ref, km_ref, oflat_ref, ow_ref):
    gm = gm_ref[...]
    tw = tw_ref[...]
    km = km_ref[...]
    kk = k_ref[0]

    gs = jnp.sum(gm, axis=-1, keepdims=True)
    need = (gs < kk).astype(jnp.int32)
    keep = (gm + km * need) > 0

    bi = lax.bitcast_convert_type(tw, jnp.int32)
    key = jnp.where(bi >= 0, bi, _INT_MIN - bi)
    key = jnp.where(keep, key, _INT_MIN)

    idx = lax.broadcasted_iota(jnp.int32, key.shape, 1)
    rowflat = (rowbase + pl.program_id(0) * r +
               lax.broadcasted_iota(jnp.int32, (r, 1), 0)) * _L

    for t in range(_K):
        m = jnp.max(key, axis=-1, keepdims=True)
        cand = jnp.where(key == m, idx, jnp.int32(_L))
        j = jnp.min(cand, axis=-1, keepdims=True)
        oflat_ref[:, t : t + 1] = rowflat + j
        wb = jnp.where(m >= 0, m, _INT_MIN - m)
        ow_ref[:, t : t + 1] = lax.bitcast_convert_type(wb, jnp.float32)
        key = jnp.where(idx == j, _INT_MIN, key)

    v = ow_ref[...]
    mx = jnp.max(v, axis=-1, keepdims=True)
    e = jnp.exp(v - mx)
    ow_ref[...] = e / jnp.sum(e, axis=-1, keepdims=True)


def _topk_call(gate_mask, token_weight, keep_k_modifier, karr, rowbase):
    b, l = gate_mask.shape
    r = min(2048, b)
    row_spec = pl.BlockSpec((r, l), lambda i: (i, 0))
    km_spec = pl.BlockSpec((1, l), lambda i: (0, 0))
    out_spec = pl.BlockSpec((r, _K), lambda i: (i, 0))
    return pl.pallas_call(
        functools.partial(_topk_body, r, rowbase),
        grid=(b // r,),
        in_specs=[
            pl.BlockSpec(memory_space=pltpu.SMEM),
            row_spec, row_spec, km_spec,
        ],
        out_specs=[out_spec, out_spec],
        out_shape=[
            jax.ShapeDtypeStruct((b, _K), jnp.int32),
            jax.ShapeDtypeStruct((b, _K), jnp.float32),
        ],
        compiler_params=pltpu.CompilerParams(
            dimension_semantics=("parallel",),
        ),
    )(karr, gate_mask, token_weight, keep_k_modifier)


def _make_sc_gather(n_total, nw):
    per_w = n_total // nw
    mesh = plsc.VectorSubcoreMesh(
        core_axis_name="c", subcore_axis_name="s",
        num_cores=_SC_CORES, num_subcores=_SC_SUBCORES)
    nc = mesh.num_cores

    @functools.partial(
        pl.kernel,
        out_type=[
            jax.ShapeDtypeStruct((nw, per_w), jnp.int32),
            jax.ShapeDtypeStruct((nw, per_w), jnp.int32),
        ],
        mesh=mesh,
        scratch_types=[
            pltpu.VMEM((per_w,), jnp.int32),
            pltpu.VMEM((per_w,), jnp.int32),
            pltpu.VMEM((per_w,), jnp.int32),
            pltpu.SemaphoreType.DMA,
        ],
    )
    def sc_gather(tok_hbm, am_hbm, idx_hbm, otok_hbm, oam_hbm,
                  idx_v, tok_v, am_v, sem):
        wid = lax.axis_index("s") * nc + lax.axis_index("c")
        pltpu.sync_copy(idx_hbm.at[wid], idx_v)
        pltpu.async_copy(tok_hbm.at[idx_v], tok_v, sem).wait()
        pltpu.async_copy(am_hbm.at[idx_v], am_v, sem).wait()
        pltpu.sync_copy(tok_v, otok_hbm.at[wid])
        pltpu.sync_copy(am_v, oam_hbm.at[wid])

    return sc_gather


def kernel(token_id, attn_mask, gate_mask, token_weight, keep_k_modifier, k):
    b, l = token_id.shape
    assert l == _L
    karr = jnp.asarray(k, jnp.int32).reshape(1)

    nw = 32
    chunks = _CHUNKS if b % (_CHUNKS * 256) == 0 else 1
    bc = b // chunks
    tok_flat = token_id.reshape(b * l)
    am_flat = attn_mask.reshape(b * l)
    gather = _make_sc_gather(bc * _K, nw)

    otoks, oams, ows = [], [], []
    for c in range(chunks):
        sl = slice(c * bc, (c + 1) * bc)
        oflat, ow = _topk_call(
            gate_mask[sl], token_weight[sl], keep_k_modifier, karr, c * bc)
        idx2 = oflat.reshape(nw, bc * _K // nw)
        otok, oam = gather(tok_flat, am_flat, idx2)
        otoks.append(otok.reshape(bc, _K))
        oams.append(oam.reshape(bc, _K))
        ows.append(ow)
    if chunks == 1:
        return (otoks[0], oams[0], ows[0])
    return (jnp.concatenate(otoks), jnp.concatenate(oams),
            jnp.concatenate(ows))


# speedup vs baseline: 1.1846x; 1.1846x over previous
import functools

import jax
import jax.numpy as jnp
import numpy as np
from jax import lax
from jax.experimental import pallas as pl
from jax.experimental.pallas import tpu as pltpu
from jax.experimental.pallas import tpu_sc as plsc

_L = 512
_K = 32
_INT_MIN = np.int32(-(2**31))
_SC_CORES = 2
_SC_SUBCORES = 16
_CHUNKS = 1


def _topk_body(r, rowbase, k_ref, gm_ref, tw_ref, km_ref, oflat_ref, ow_ref):
    gm = gm_ref[...]
    tw = tw_ref[...]
    km = km_ref[...]
    kk = k_ref[0]

    gs = jnp.sum(gm, axis=-1, keepdims=True)
    need = (gs < kk).astype(jnp.int32)
    keep = (gm + km * need) > 0

    bi = lax.bitcast_convert_type(tw, jnp.int32)
    key = jnp.where(bi >= 0, bi, _INT_MIN - bi)
    key = jnp.where(keep, key, _INT_MIN)

    idx = lax.broadcasted_iota(jnp.int32, key.shape, 1)
    rowflat = (rowbase + pl.program_id(0) * r +
               lax.broadcasted_iota(jnp.int32, (r, 1), 0)) * _L

    for t in range(_K):
        m = jnp.max(key, axis=-1, keepdims=True)
        cand = jnp.where(key == m, idx, jnp.int32(_L))
        j = jnp.min(cand, axis=-1, keepdims=True)
        oflat_ref[:, t : t + 1] = rowflat + j
        wb = jnp.where(m >= 0, m, _INT_MIN - m)
        ow_ref[:, t : t + 1] = lax.bitcast_convert_type(wb, jnp.float32)
        key = jnp.where(idx == j, _INT_MIN, key)

    v = ow_ref[...]
    mx = jnp.max(v, axis=-1, keepdims=True)
    e = jnp.exp(v - mx)
    ow_ref[...] = e / jnp.sum(e, axis=-1, keepdims=True)


def _topk_call(gate_mask, token_weight, keep_k_modifier, karr, rowbase):
    b, l = gate_mask.shape
    r = min(1024, b)
    row_spec = pl.BlockSpec((r, l), lambda i: (i, 0))
    km_spec = pl.BlockSpec((1, l), lambda i: (0, 0))
    out_spec = pl.BlockSpec((r, _K), lambda i: (i, 0))
    return pl.pallas_call(
        functools.partial(_topk_body, r, rowbase),
        grid=(b // r,),
        in_specs=[
            pl.BlockSpec(memory_space=pltpu.SMEM),
            row_spec, row_spec, km_spec,
        ],
        out_specs=[out_spec, out_spec],
        out_shape=[
            jax.ShapeDtypeStruct((b, _K), jnp.int32),
            jax.ShapeDtypeStruct((b, _K), jnp.float32),
        ],
        compiler_params=pltpu.CompilerParams(
            dimension_semantics=("parallel",),
        ),
    )(karr, gate_mask, token_weight, keep_k_modifier)


def _make_sc_gather(n_total, nw):
    per_w = n_total // nw
    mesh = plsc.VectorSubcoreMesh(
        core_axis_name="c", subcore_axis_name="s",
        num_cores=_SC_CORES, num_subcores=_SC_SUBCORES)
    nc = mesh.num_cores

    @functools.partial(
        pl.kernel,
        out_type=[
            jax.ShapeDtypeStruct((nw, per_w), jnp.int32),
            jax.ShapeDtypeStruct((nw, per_w), jnp.int32),
        ],
        mesh=mesh,
        scratch_types=[
            pltpu.VMEM((per_w,), jnp.int32),
            pltpu.VMEM((per_w,), jnp.int32),
            pltpu.VMEM((per_w,), jnp.int32),
            pltpu.SemaphoreType.DMA,
        ],
    )
    def sc_gather(tok_hbm, am_hbm, idx_hbm, otok_hbm, oam_hbm,
                  idx_v, tok_v, am_v, sem):
        wid = lax.axis_index("s") * nc + lax.axis_index("c")
        pltpu.sync_copy(idx_hbm.at[wid], idx_v)
        cp_tok = pltpu.async_copy(tok_hbm.at[idx_v], tok_v, sem)
        cp_am = pltpu.async_copy(am_hbm.at[idx_v], am_v, sem)
        cp_tok.wait()
        cp_am.wait()
        pltpu.sync_copy(tok_v, otok_hbm.at[wid])
        pltpu.sync_copy(am_v, oam_hbm.at[wid])

    return sc_gather


def kernel(token_id, attn_mask, gate_mask, token_weight, keep_k_modifier, k):
    b, l = token_id.shape
    assert l == _L
    karr = jnp.asarray(k, jnp.int32).reshape(1)

    nw = 32
    chunks = _CHUNKS if b % (_CHUNKS * 256) == 0 else 1
    bc = b // chunks
    tok_flat = token_id.reshape(b * l)
    am_flat = attn_mask.reshape(b * l)
    gather = _make_sc_gather(bc * _K, nw)

    otoks, oams, ows = [], [], []
    for c in range(chunks):
        sl = slice(c * bc, (c + 1) * bc)
        oflat, ow = _topk_call(
            gate_mask[sl], token_weight[sl], keep_k_modifier, karr, c * bc)
        idx2 = oflat.reshape(nw, bc * _K // nw)
        otok, oam = gather(tok_flat, am_flat, idx2)
        otoks.append(otok.reshape(bc, _K))
        oams.append(oam.reshape(bc, _K))
        ows.append(ow)
    if chunks == 1:
        return (otoks[0], oams[0], ows[0])
    return (jnp.concatenate(otoks), jnp.concatenate(oams),
            jnp.concatenate(ows))
